# baseline (speedup 1.0000x reference)
import functools

import jax
import jax.numpy as jnp
from jax import lax
from jax.experimental import pallas as pl
from jax.experimental.pallas import tpu as pltpu
from jax.experimental.pallas import tpu_sc as plsc

_MAX_REL = 32
_NTAB = 2 * _MAX_REL + 1


def _sc_rpe(table_hbm, out_hbm, tab_v, buf_v, g_sh, sem, *, seq_len, d_model,
            num_workers, rep_rows, buf_rows):
    cid = lax.axis_index("c")
    sid = lax.axis_index("s")
    num_cores = 2
    wid = cid * (num_workers // num_cores) + sid
    rows_per_w = seq_len // num_workers
    row_words = seq_len * d_model

    @pl.when(sid == 0)
    def _build_g():
        pltpu.sync_copy(table_hbm, tab_v)

        def _fill(dst_row0):
            def body(r, _):
                for v in range(d_model // 16):
                    buf_v[pl.ds(r * d_model + v * 16, 16)] = (
                        tab_v[pl.ds(dst_row0 * d_model + v * 16, 16)])
                return 0
            lax.fori_loop(0, buf_rows, body, 0)

        _fill(0)
        for c in range(rep_rows // buf_rows):
            pltpu.sync_copy(
                buf_v, g_sh.at[pl.ds(c * buf_rows * d_model,
                                     buf_rows * d_model)])
        pltpu.sync_copy(
            tab_v.at[pl.ds(d_model, (_NTAB - 1) * d_model)],
            g_sh.at[pl.ds(rep_rows * d_model, (_NTAB - 1) * d_model)])
        _fill(_NTAB - 1)
        hi0 = rep_rows + _NTAB - 1
        for c in range((2 * seq_len - hi0) // buf_rows):
            pltpu.sync_copy(
                buf_v, g_sh.at[pl.ds((hi0 + c * buf_rows) * d_model,
                                     buf_rows * d_model)])

    plsc.subcore_barrier()

    copies = []
    for r in range(rows_per_w):
        i = wid * rows_per_w + r
        off = (seq_len - 1 - i) * d_model
        copies.append(pltpu.async_copy(
            g_sh.at[pl.ds(off, row_words)], out_hbm.at[i], sem))
    for c in copies:
        c.wait()


def kernel(x, table):
    seq_len = x.shape[1]
    d_model = table.shape[1]
    num_workers = 32
    rep_rows = seq_len - _MAX_REL
    buf_rows = rep_rows // 2

    mesh = plsc.VectorSubcoreMesh(core_axis_name="c", subcore_axis_name="s")
    body = functools.partial(
        _sc_rpe, seq_len=seq_len, d_model=d_model, num_workers=num_workers,
        rep_rows=rep_rows, buf_rows=buf_rows)

    run = pl.kernel(
        body,
        mesh=mesh,
        out_type=jax.ShapeDtypeStruct((seq_len, seq_len * d_model),
                                      jnp.float32),
        scratch_types=[
            pltpu.VMEM((_NTAB * d_model,), jnp.float32),
            pltpu.VMEM((buf_rows * d_model,), jnp.float32),
            pltpu.VMEM_SHARED((2 * seq_len * d_model,), jnp.float32),
            pltpu.SemaphoreType.DMA,
        ],
    )
    rel = run(table.reshape(_NTAB * d_model))
    return (x, rel.reshape(seq_len, seq_len, d_model))


# speedup vs baseline: 2.3682x; 2.3682x over previous
import functools

import jax
import jax.numpy as jnp
from jax import lax
from jax.experimental import pallas as pl
from jax.experimental.pallas import tpu as pltpu
from jax.experimental.pallas import tpu_sc as plsc

_MAX_REL = 32
_NTAB = 2 * _MAX_REL + 1


def _sc_expand(table_hbm, out_hbm, tab_v, buf_v, e_sh, sem, *, seq_len,
               d_model, e_rows, band_rows, rep_rows, buf_rows):
    cid = lax.axis_index("c")
    sid = lax.axis_index("s")

    @pl.when(sid == 0)
    def _build_e():
        pltpu.sync_copy(table_hbm, tab_v)

        def _fill(src_row):
            def body(r, _):
                for v in range(d_model // 16):
                    buf_v[pl.ds(r * d_model + v * 16, 16)] = (
                        tab_v[pl.ds(src_row * d_model + v * 16, 16)])
                return 0
            lax.fori_loop(0, buf_rows, body, 0)

        _fill(0)
        for c in range(rep_rows // buf_rows):
            pltpu.sync_copy(
                buf_v, e_sh.at[pl.ds(c * buf_rows * d_model,
                                     buf_rows * d_model)])
        pltpu.sync_copy(
            tab_v.at[pl.ds(d_model, (_NTAB - 1) * d_model)],
            e_sh.at[pl.ds(rep_rows * d_model, (_NTAB - 1) * d_model)])
        _fill(_NTAB - 1)
        hi0 = rep_rows + _NTAB - 1
        tail = e_rows - hi0
        for c in range(tail // buf_rows):
            pltpu.sync_copy(
                buf_v, e_sh.at[pl.ds((hi0 + c * buf_rows) * d_model,
                                     buf_rows * d_model)])
        rem = tail % buf_rows
        if rem:
            pltpu.sync_copy(
                buf_v.at[pl.ds(0, rem * d_model)],
                e_sh.at[pl.ds((e_rows - rem) * d_model, rem * d_model)])

    plsc.subcore_barrier()

    @pl.when(sid < 4)
    def _emit():
        k = (sid * 2 + cid) * d_model
        pltpu.async_copy(
            e_sh.at[pl.ds(k, band_rows * d_model)],
            out_hbm.at[sid * 2 + cid], sem).wait()


def _tc_stream(gs_ref, out_ref, *, seq_len, d_model, rows_per_blk):
    q = seq_len - rows_per_blk * (pl.program_id(0) + 1)
    q = pl.multiple_of(q, 8)
    for r in range(rows_per_blk):
        out_ref[r, :, :] = gs_ref[7 - r, pl.ds(q, seq_len), :]


def kernel(x, table):
    seq_len = x.shape[1]
    d_model = table.shape[1]
    rows_per_blk = 8
    band_rows = 2 * seq_len
    e_rows = band_rows + 16
    rep_rows = seq_len - _MAX_REL
    buf_rows = rep_rows // 2

    mesh = plsc.VectorSubcoreMesh(core_axis_name="c", subcore_axis_name="s")
    sc_body = functools.partial(
        _sc_expand, seq_len=seq_len, d_model=d_model, e_rows=e_rows,
        band_rows=band_rows, rep_rows=rep_rows, buf_rows=buf_rows)
    gs = pl.kernel(
        sc_body,
        mesh=mesh,
        out_type=jax.ShapeDtypeStruct((8, band_rows * d_model), jnp.float32),
        scratch_types=[
            pltpu.VMEM((_NTAB * d_model,), jnp.float32),
            pltpu.VMEM((buf_rows * d_model,), jnp.float32),
            pltpu.VMEM_SHARED((e_rows * d_model,), jnp.float32),
            pltpu.SemaphoreType.DMA,
        ],
    )(table.reshape(_NTAB * d_model))
    gs = gs.reshape(8, band_rows, d_model)

    tc_body = functools.partial(
        _tc_stream, seq_len=seq_len, d_model=d_model,
        rows_per_blk=rows_per_blk)
    rel = pl.pallas_call(
        tc_body,
        grid=(seq_len // rows_per_blk,),
        in_specs=[pl.BlockSpec((8, band_rows, d_model), lambda i: (0, 0, 0))],
        out_specs=pl.BlockSpec((rows_per_blk, seq_len, d_model),
                               lambda i: (i, 0, 0)),
        out_shape=jax.ShapeDtypeStruct((seq_len, seq_len, d_model),
                                       jnp.float32),
    )(gs)
    return (x, rel)


# speedup vs baseline: 3.2946x; 1.3912x over previous
import functools

import jax
import jax.numpy as jnp
from jax import lax
from jax.experimental import pallas as pl
from jax.experimental.pallas import tpu as pltpu
from jax.experimental.pallas import tpu_sc as plsc

_MAX_REL = 32
_NTAB = 2 * _MAX_REL + 1


def _sc_expand(table_hbm, out_hbm, tab_v, buf_v, *, seq_len, d_model,
               seg_rows):
    wid = lax.axis_index("c") * 16 + lax.axis_index("s")
    base = wid * seg_rows
    pltpu.sync_copy(table_hbm, tab_v)
    lo = seq_len - 1 - _MAX_REL
    for r in range(seg_rows):
        src = jnp.clip(base + r - lo, 0, _NTAB - 1) * d_model
        for v in range(d_model // 16):
            buf_v[pl.ds(r * d_model + v * 16, 16)] = (
                tab_v[pl.ds(src + v * 16, 16)])
    pltpu.sync_copy(
        buf_v, out_hbm.at[pl.ds(base * d_model, seg_rows * d_model)])


def _tc_stream(e_ref, out_ref, g_ref, *, seq_len, d_model, rows_per_blk):
    @pl.when(pl.program_id(0) == 0)
    def _build_g():
        for k in range(8):
            g_ref[k, :, :] = e_ref[k:k + 2 * seq_len, :]

    q = seq_len - rows_per_blk * (pl.program_id(0) + 1)
    q = pl.multiple_of(q, 8)
    for r in range(rows_per_blk):
        out_ref[r, :, :] = g_ref[7 - r, pl.ds(q, seq_len), :]


def kernel(x, table):
    seq_len = x.shape[1]
    d_model = table.shape[1]
    rows_per_blk = 8
    num_workers = 32
    e_rows = 2 * seq_len + num_workers
    seg_rows = e_rows // num_workers

    mesh = plsc.VectorSubcoreMesh(core_axis_name="c", subcore_axis_name="s")
    sc_body = functools.partial(
        _sc_expand, seq_len=seq_len, d_model=d_model, seg_rows=seg_rows)
    e = pl.kernel(
        sc_body,
        mesh=mesh,
        out_type=jax.ShapeDtypeStruct((e_rows * d_model,), jnp.float32),
        scratch_types=[
            pltpu.VMEM((_NTAB * d_model,), jnp.float32),
            pltpu.VMEM((seg_rows * d_model,), jnp.float32),
        ],
    )(table.reshape(_NTAB * d_model))
    e = e.reshape(e_rows, d_model)

    tc_body = functools.partial(
        _tc_stream, seq_len=seq_len, d_model=d_model,
        rows_per_blk=rows_per_blk)
    rel = pl.pallas_call(
        tc_body,
        grid=(seq_len // rows_per_blk,),
        in_specs=[pl.BlockSpec((e_rows, d_model), lambda i: (0, 0))],
        out_specs=pl.BlockSpec((rows_per_blk, seq_len, d_model),
                               lambda i: (i, 0, 0)),
        out_shape=jax.ShapeDtypeStruct((seq_len, seq_len, d_model),
                                       jnp.float32),
        scratch_shapes=[pltpu.VMEM((8, 2 * seq_len, d_model), jnp.float32)],
    )(e)
    return (x, rel)
